# baseline (speedup 1.0000x reference)
import jax
import jax.numpy as jnp
from jax.experimental import pallas as pl
from jax.experimental.pallas import tpu as pltpu

_N, _D = 10000, 128
_C = 1024
_NC = _N // _C
_T = _N - _NC * _C


def _heads_body(x_hbm, cons_hbm, obj_hbm,
                xb2, cons2, obj2, xb_t, cons_t, obj_t,
                in_sems, cons_sems, obj_sems, tin, tcons, tobj):

    def in_copy(i):
        if i < _NC:
            return pltpu.make_async_copy(
                x_hbm.at[pl.ds(i * _C, _C), :], xb2.at[i % 2],
                in_sems.at[i % 2])
        return pltpu.make_async_copy(
            x_hbm.at[pl.ds(_NC * _C, _T), :], xb_t, tin)

    def cons_copy(i):
        if i < _NC:
            return pltpu.make_async_copy(
                cons2.at[i % 2], cons_hbm.at[pl.ds(i * _C, _C), :],
                cons_sems.at[i % 2])
        return pltpu.make_async_copy(
            cons_t, cons_hbm.at[pl.ds(_NC * _C, _T), :], tcons)

    def obj_copy(i):
        if i < _NC:
            return pltpu.make_async_copy(
                obj2.at[i % 2], obj_hbm.at[:, pl.ds(i * _C, _C)],
                obj_sems.at[i % 2])
        return pltpu.make_async_copy(
            obj_t, obj_hbm.at[:, pl.ds(_NC * _C, _T)], tobj)

    n_steps = _NC + 1
    in_copy(0).start()
    for i in range(n_steps):
        if i + 1 < n_steps:
            in_copy(i + 1).start()
        in_copy(i).wait()
        xb = xb2[i % 2] if i < _NC else xb_t[:]
        m = jnp.max(xb, axis=1, keepdims=True)
        e = jnp.exp(xb - m)
        s = jnp.sum(e, axis=1, keepdims=True)
        c = e / s
        t = jax.nn.sigmoid(xb.T)
        if i >= 2:
            cons_copy(i - 2).wait()
            obj_copy(i - 2).wait()
        if i < _NC:
            cons2[i % 2] = c
            obj2[i % 2] = t
        else:
            cons_t[:] = c
            obj_t[:] = t
        cons_copy(i).start()
        obj_copy(i).start()
    cons_copy(n_steps - 2).wait()
    obj_copy(n_steps - 2).wait()
    cons_copy(n_steps - 1).wait()
    obj_copy(n_steps - 1).wait()


def kernel(x, graph, edge_index):
    del graph, edge_index
    n, d = x.shape
    cons, obj = pl.pallas_call(
        _heads_body,
        in_specs=[pl.BlockSpec(memory_space=pl.ANY)],
        out_specs=[
            pl.BlockSpec(memory_space=pl.ANY),
            pl.BlockSpec(memory_space=pl.ANY),
        ],
        out_shape=[
            jax.ShapeDtypeStruct((n, d), x.dtype),
            jax.ShapeDtypeStruct((d, n), x.dtype),
        ],
        scratch_shapes=[
            pltpu.VMEM((2, _C, _D), jnp.float32),
            pltpu.VMEM((2, _C, _D), jnp.float32),
            pltpu.VMEM((2, _D, _C), jnp.float32),
            pltpu.VMEM((_T, _D), jnp.float32),
            pltpu.VMEM((_T, _D), jnp.float32),
            pltpu.VMEM((_D, _T), jnp.float32),
            pltpu.SemaphoreType.DMA((2,)),
            pltpu.SemaphoreType.DMA((2,)),
            pltpu.SemaphoreType.DMA((2,)),
            pltpu.SemaphoreType.DMA,
            pltpu.SemaphoreType.DMA,
            pltpu.SemaphoreType.DMA,
        ],
    )(x)
    return (cons, obj)


# speedup vs baseline: 1.2468x; 1.2468x over previous
import jax
import jax.numpy as jnp
from jax.experimental import pallas as pl
from jax.experimental.pallas import tpu as pltpu

_N, _D = 10000, 128
_C = 1024
_NC = _N // _C
_T = _N - _NC * _C
_STEPS = _NC + 1


def _chunk(i):
    return (i * _C, _C) if i < _NC else (_NC * _C, _T)


def _heads_body(x_hbm, cons_hbm, obj_hbm,
                xv, cv, ov, in_sems, cons_sems, obj_sems):

    def in_copy(i):
        off, sz = _chunk(i)
        return pltpu.make_async_copy(
            x_hbm.at[pl.ds(off, sz), :], xv.at[pl.ds(off, sz), :],
            in_sems.at[i])

    def cons_copy(i):
        off, sz = _chunk(i)
        return pltpu.make_async_copy(
            cv.at[pl.ds(off, sz), :], cons_hbm.at[pl.ds(off, sz), :],
            cons_sems.at[i])

    def obj_copy(i):
        off, sz = _chunk(i)
        return pltpu.make_async_copy(
            ov.at[:, pl.ds(off, sz)], obj_hbm.at[:, pl.ds(off, sz)],
            obj_sems.at[i])

    for i in range(_STEPS):
        in_copy(i).start()
    for i in range(_STEPS):
        off, sz = _chunk(i)
        in_copy(i).wait()
        xb = xv[pl.ds(off, sz), :]
        m = jnp.max(xb, axis=1, keepdims=True)
        e = jnp.exp(xb - m)
        s = jnp.sum(e, axis=1, keepdims=True)
        cv[pl.ds(off, sz), :] = e / s
        ov[:, pl.ds(off, sz)] = jax.nn.sigmoid(xb.T)
        cons_copy(i).start()
        obj_copy(i).start()
    for i in range(_STEPS):
        cons_copy(i).wait()
        obj_copy(i).wait()


def kernel(x, graph, edge_index):
    del graph, edge_index
    n, d = x.shape
    cons, obj = pl.pallas_call(
        _heads_body,
        in_specs=[pl.BlockSpec(memory_space=pl.ANY)],
        out_specs=[
            pl.BlockSpec(memory_space=pl.ANY),
            pl.BlockSpec(memory_space=pl.ANY),
        ],
        out_shape=[
            jax.ShapeDtypeStruct((n, d), x.dtype),
            jax.ShapeDtypeStruct((d, n), x.dtype),
        ],
        scratch_shapes=[
            pltpu.VMEM((_N, _D), jnp.float32),
            pltpu.VMEM((_N, _D), jnp.float32),
            pltpu.VMEM((_D, _N), jnp.float32),
            pltpu.SemaphoreType.DMA((_STEPS,)),
            pltpu.SemaphoreType.DMA((_STEPS,)),
            pltpu.SemaphoreType.DMA((_STEPS,)),
        ],
    )(x)
    return (cons, obj)
